# baseline (speedup 1.0000x reference)
import functools

import jax
import jax.numpy as jnp
from jax import lax
from jax.experimental import pallas as pl
from jax.experimental.pallas import tpu as pltpu
from jax.experimental.pallas import tpu_sc as plsc

N = 10000
E = 320000
D = 128
NC = 2
NS = 16
HALF = N // NC
ACC_R = 5008
EPT = E // NS
CHUNK = 125
NCHUNK = EPT // CHUNK
RPT = ACC_R // NS

_sc_mesh = plsc.VectorSubcoreMesh(core_axis_name="c", subcore_axis_name="s")


@functools.partial(
    pl.kernel,
    out_type=jax.ShapeDtypeStruct((NC, NS, RPT, D), jnp.float32),
    mesh=_sc_mesh,
    scratch_types=[
        pltpu.VMEM((NCHUNK, CHUNK), jnp.int32),
        pltpu.VMEM((NCHUNK, CHUNK), jnp.int32),
        pltpu.VMEM((2, CHUNK, D), jnp.float32),
        pltpu.VMEM_SHARED((ACC_R, D), jnp.float32),
        pltpu.SemaphoreType.DMA,
        pltpu.SemaphoreType.DMA,
    ],
)
def _sc_segment_sum(x_hbm, src_hbm, dst_hbm, zeros_hbm, out_hbm,
                    src_v, dst_v, rows_v, acc, sem0, sem1):
    cid = lax.axis_index("c")
    sid = lax.axis_index("s")

    pltpu.sync_copy(zeros_hbm, acc.at[pl.ds(sid * RPT, RPT)])
    pltpu.sync_copy(src_hbm.at[cid, sid], src_v)
    pltpu.sync_copy(dst_hbm.at[cid, sid], dst_v)
    plsc.subcore_barrier()

    pltpu.async_copy(x_hbm.at[src_v.at[0]], rows_v.at[0], sem0)

    def body(jj, _):
        j0 = 2 * jj
        j1 = j0 + 1
        pltpu.async_copy(x_hbm.at[src_v.at[j1]], rows_v.at[1], sem1)
        pltpu.make_async_copy(x_hbm.at[src_v.at[j0]], rows_v.at[0], sem0).wait()
        pltpu.sync_copy(rows_v.at[0], acc.at[dst_v.at[j0]], add=True)

        @pl.when(jj + 1 < NCHUNK // 2)
        def _prefetch():
            pltpu.async_copy(x_hbm.at[src_v.at[j0 + 2]], rows_v.at[0], sem0)

        pltpu.make_async_copy(x_hbm.at[src_v.at[j1]], rows_v.at[1], sem1).wait()
        pltpu.sync_copy(rows_v.at[1], acc.at[dst_v.at[j1]], add=True)
        return 0

    lax.fori_loop(0, NCHUNK // 2, body, 0)

    plsc.subcore_barrier()
    pltpu.sync_copy(acc.at[pl.ds(sid * RPT, RPT)], out_hbm.at[cid, sid])


def _mlp1_body(x_ref, agg_ref, w1_ref, b1_ref, w2_ref, b2_ref, o_ref):
    h = x_ref[...] + agg_ref[...]
    h = jnp.dot(h, w1_ref[...], preferred_element_type=jnp.float32) + b1_ref[...]
    h = jnp.maximum(h, 0.0)
    h = jnp.dot(h, w2_ref[...], preferred_element_type=jnp.float32) + b2_ref[...]
    o_ref[...] = jnp.maximum(h, 0.0)


def _mlp2_body(x_ref, agg_ref, w3_ref, b3_ref, w4_ref, b4_ref, o_ref):
    h = x_ref[...] + agg_ref[...]
    h = jnp.dot(h, w3_ref[...], preferred_element_type=jnp.float32) + b3_ref[...]
    h = jnp.maximum(h, 0.0)
    h = jnp.dot(h, w4_ref[...], preferred_element_type=jnp.float32) + b4_ref[...]
    m = jnp.max(h, axis=1, keepdims=True)
    s = jnp.sum(jnp.exp(h - m), axis=1, keepdims=True)
    o_ref[...] = h - m - jnp.log(s)


_mlp1 = pl.pallas_call(
    _mlp1_body, out_shape=jax.ShapeDtypeStruct((N, D), jnp.float32))
_mlp2 = pl.pallas_call(
    _mlp2_body, out_shape=jax.ShapeDtypeStruct((N, D), jnp.float32))


def _agg_from_out(out):
    return out.reshape(NC, ACC_R, D)[:, :HALF].reshape(N, D)


@jax.jit
def kernel(x, edge_index, W1, b1, W2, b2, W3, b3, W4, b4):
    src = edge_index[0]
    dst = edge_index[1]
    lane = jnp.arange(E, dtype=jnp.int32)
    in0 = dst < HALF
    dummy = HALF + (lane & 7)
    src_loc = jnp.stack([
        jnp.where(in0, src, N),
        jnp.where(in0, N, src),
    ]).reshape(NC, NS, NCHUNK, CHUNK)
    dst_loc = jnp.stack([
        jnp.where(in0, dst, dummy),
        jnp.where(in0, dummy, dst - HALF),
    ]).reshape(NC, NS, NCHUNK, CHUNK)
    zeros = jnp.zeros((RPT, D), jnp.float32)
    xp = jnp.concatenate([x, jnp.zeros((8, D), jnp.float32)])
    agg1 = _agg_from_out(_sc_segment_sum(xp, src_loc, dst_loc, zeros))
    h1 = _mlp1(x, agg1, W1, b1.reshape(1, D), W2, b2.reshape(1, D))
    h1p = jnp.concatenate([h1, jnp.zeros((8, D), jnp.float32)])
    agg2 = _agg_from_out(_sc_segment_sum(h1p, src_loc, dst_loc, zeros))
    return _mlp2(h1, agg2, W3, b3.reshape(1, D), W4, b4.reshape(1, D))


# speedup vs baseline: 52.2977x; 52.2977x over previous
import functools

import jax
import jax.numpy as jnp
from jax import lax
from jax.experimental import pallas as pl
from jax.experimental.pallas import tpu as pltpu
from jax.experimental.pallas import tpu_sc as plsc

N = 10000
E = 320000
D = 128
NC = 2
NS = 16
HALF = N // NC
ACC_R = 5008
EPT = E // NS
CHUNK = 125
NCHUNK = EPT // CHUNK
RPT = ACC_R // NS

_sc_mesh = plsc.VectorSubcoreMesh(core_axis_name="c", subcore_axis_name="s")


@functools.partial(
    pl.kernel,
    out_type=jax.ShapeDtypeStruct((NC, NS, RPT, D), jnp.float32),
    mesh=_sc_mesh,
    scratch_types=[
        pltpu.VMEM((NCHUNK, CHUNK), jnp.int32),
        pltpu.VMEM((NCHUNK, CHUNK), jnp.int32),
        pltpu.VMEM((2, CHUNK, D), jnp.float32),
        pltpu.VMEM_SHARED((ACC_R, D), jnp.float32),
        pltpu.SemaphoreType.DMA,
        pltpu.SemaphoreType.DMA,
    ],
)
def _sc_segment_sum(x_hbm, src_hbm, dst_hbm, zeros_hbm, out_hbm,
                    src_v, dst_v, rows_v, acc, sem0, sem1):
    cid = lax.axis_index("c")
    sid = lax.axis_index("s")

    pltpu.sync_copy(zeros_hbm, acc.at[pl.ds(sid * RPT, RPT)])
    pltpu.sync_copy(src_hbm.at[cid, sid], src_v)
    pltpu.sync_copy(dst_hbm.at[cid, sid], dst_v)
    plsc.subcore_barrier()

    pltpu.async_copy(x_hbm.at[src_v.at[0]], rows_v.at[0], sem0)

    def body(jj, _):
        j0 = 2 * jj
        j1 = j0 + 1
        pltpu.async_copy(x_hbm.at[src_v.at[j1]], rows_v.at[1], sem1)
        pltpu.make_async_copy(x_hbm.at[src_v.at[j0]], rows_v.at[0], sem0).wait()
        pltpu.sync_copy(rows_v.at[0], acc.at[dst_v.at[j0]], add=True)

        @pl.when(jj + 1 < NCHUNK // 2)
        def _prefetch():
            pltpu.async_copy(x_hbm.at[src_v.at[j0 + 2]], rows_v.at[0], sem0)

        pltpu.make_async_copy(x_hbm.at[src_v.at[j1]], rows_v.at[1], sem1).wait()
        pltpu.sync_copy(rows_v.at[1], acc.at[dst_v.at[j1]], add=True)
        return 0

    lax.fori_loop(0, NCHUNK // 2, body, 0)

    plsc.subcore_barrier()
    pltpu.sync_copy(acc.at[pl.ds(sid * RPT, RPT)], out_hbm.at[cid, sid])


def _mlp1_body(x_ref, agg_ref, w1_ref, b1_ref, w2_ref, b2_ref, o_ref):
    h = x_ref[...] + agg_ref[...]
    h = jnp.dot(h, w1_ref[...], preferred_element_type=jnp.float32) + b1_ref[...]
    h = jnp.maximum(h, 0.0)
    h = jnp.dot(h, w2_ref[...], preferred_element_type=jnp.float32) + b2_ref[...]
    o_ref[...] = jnp.maximum(h, 0.0)


def _mlp2_body(x_ref, agg_ref, w3_ref, b3_ref, w4_ref, b4_ref, o_ref):
    h = x_ref[...] + agg_ref[...]
    h = jnp.dot(h, w3_ref[...], preferred_element_type=jnp.float32) + b3_ref[...]
    h = jnp.maximum(h, 0.0)
    h = jnp.dot(h, w4_ref[...], preferred_element_type=jnp.float32) + b4_ref[...]
    m = jnp.max(h, axis=1, keepdims=True)
    s = jnp.sum(jnp.exp(h - m), axis=1, keepdims=True)
    o_ref[...] = h - m - jnp.log(s)


_mlp1 = pl.pallas_call(
    _mlp1_body, out_shape=jax.ShapeDtypeStruct((N, D), jnp.float32))
_mlp2 = pl.pallas_call(
    _mlp2_body, out_shape=jax.ShapeDtypeStruct((N, D), jnp.float32))


def _agg_from_out(out):
    return out.reshape(NC, ACC_R, D)[:, :HALF].reshape(N, D)


@jax.jit
def kernel(x, edge_index, W1, b1, W2, b2, W3, b3, W4, b4):
    src = edge_index[0]
    dst = edge_index[1]
    lane = jnp.arange(E, dtype=jnp.int32)
    in0 = dst < HALF
    dummy = HALF + (lane & 7)
    src_loc = jnp.stack([src, src]).reshape(NC, NS, NCHUNK, CHUNK)
    dst_loc = jnp.stack([
        jnp.where(in0, dst, dummy),
        jnp.where(in0, dummy, dst - HALF),
    ]).reshape(NC, NS, NCHUNK, CHUNK)
    zeros = jnp.zeros((RPT, D), jnp.float32)
    xp = jnp.concatenate([x, jnp.zeros((8, D), jnp.float32)])
    agg1 = _agg_from_out(_sc_segment_sum(xp, src_loc, dst_loc, zeros))
    h1 = _mlp1(x, agg1, W1, b1.reshape(1, D), W2, b2.reshape(1, D))
    h1p = jnp.concatenate([h1, jnp.zeros((8, D), jnp.float32)])
    agg2 = _agg_from_out(_sc_segment_sum(h1p, src_loc, dst_loc, zeros))
    return _mlp2(h1, agg2, W3, b3.reshape(1, D), W4, b4.reshape(1, D))


# speedup vs baseline: 59.7288x; 1.1421x over previous
import functools

import jax
import jax.numpy as jnp
from jax import lax
from jax.experimental import pallas as pl
from jax.experimental.pallas import tpu as pltpu
from jax.experimental.pallas import tpu_sc as plsc

N = 10000
E = 320000
D = 128
NC = 2
NS = 16
DH = D // NC
EPT = E // NS
CHUNK = 125
NCHUNK = EPT // CHUNK
RPT = N // NS

_sc_mesh = plsc.VectorSubcoreMesh(core_axis_name="c", subcore_axis_name="s")


@functools.partial(
    pl.kernel,
    out_type=jax.ShapeDtypeStruct((NC, NS, RPT, DH), jnp.float32),
    mesh=_sc_mesh,
    compiler_params=pltpu.CompilerParams(use_tc_tiling_on_sc=False),
    scratch_types=[
        pltpu.VMEM((NCHUNK, CHUNK), jnp.int32),
        pltpu.VMEM((NCHUNK, CHUNK), jnp.int32),
        pltpu.VMEM((2, CHUNK, DH), jnp.float32),
        pltpu.VMEM_SHARED((N, DH), jnp.float32),
        pltpu.SemaphoreType.DMA,
        pltpu.SemaphoreType.DMA,
    ],
)
def _sc_segment_sum(x_hbm, src_hbm, dst_hbm, zeros_hbm, out_hbm,
                    src_v, dst_v, rows_v, acc, sem0, sem1):
    cid = lax.axis_index("c")
    sid = lax.axis_index("s")

    pltpu.sync_copy(zeros_hbm, acc.at[pl.ds(sid * RPT, RPT)])
    pltpu.sync_copy(src_hbm.at[cid, sid], src_v)
    pltpu.sync_copy(dst_hbm.at[sid], dst_v)
    plsc.subcore_barrier()

    pltpu.async_copy(x_hbm.at[src_v.at[0]], rows_v.at[0], sem0)

    def body(jj, _):
        j0 = 2 * jj
        j1 = j0 + 1
        pltpu.async_copy(x_hbm.at[src_v.at[j1]], rows_v.at[1], sem1)
        pltpu.make_async_copy(x_hbm.at[src_v.at[j0]], rows_v.at[0], sem0).wait()
        pltpu.sync_copy(rows_v.at[0], acc.at[dst_v.at[j0]], add=True)

        @pl.when(jj + 1 < NCHUNK // 2)
        def _prefetch():
            pltpu.async_copy(x_hbm.at[src_v.at[j0 + 2]], rows_v.at[0], sem0)

        pltpu.make_async_copy(x_hbm.at[src_v.at[j1]], rows_v.at[1], sem1).wait()
        pltpu.sync_copy(rows_v.at[1], acc.at[dst_v.at[j1]], add=True)
        return 0

    lax.fori_loop(0, NCHUNK // 2, body, 0)

    plsc.subcore_barrier()
    pltpu.sync_copy(acc.at[pl.ds(sid * RPT, RPT)], out_hbm.at[cid, sid])


def _mlp1_body(x_ref, agg_ref, w1_ref, b1_ref, w2_ref, b2_ref, o_ref):
    h = x_ref[...] + agg_ref[...]
    h = jnp.dot(h, w1_ref[...], preferred_element_type=jnp.float32) + b1_ref[...]
    h = jnp.maximum(h, 0.0)
    h = jnp.dot(h, w2_ref[...], preferred_element_type=jnp.float32) + b2_ref[...]
    o_ref[...] = jnp.maximum(h, 0.0)


def _mlp2_body(x_ref, agg_ref, w3_ref, b3_ref, w4_ref, b4_ref, o_ref):
    h = x_ref[...] + agg_ref[...]
    h = jnp.dot(h, w3_ref[...], preferred_element_type=jnp.float32) + b3_ref[...]
    h = jnp.maximum(h, 0.0)
    h = jnp.dot(h, w4_ref[...], preferred_element_type=jnp.float32) + b4_ref[...]
    m = jnp.max(h, axis=1, keepdims=True)
    s = jnp.sum(jnp.exp(h - m), axis=1, keepdims=True)
    o_ref[...] = h - m - jnp.log(s)


_mlp1 = pl.pallas_call(
    _mlp1_body, out_shape=jax.ShapeDtypeStruct((N, D), jnp.float32))
_mlp2 = pl.pallas_call(
    _mlp2_body, out_shape=jax.ShapeDtypeStruct((N, D), jnp.float32))


def _split_rows(h):
    return jnp.concatenate([h[:, :DH], h[:, DH:]], axis=0)


def _agg_from_out(out):
    o = out.reshape(NC, N, DH)
    return jnp.concatenate([o[0], o[1]], axis=1)


@jax.jit
def kernel(x, edge_index, W1, b1, W2, b2, W3, b3, W4, b4):
    src = edge_index[0]
    dst = edge_index[1].reshape(NS, NCHUNK, CHUNK)
    src2 = jnp.stack([src, src + N]).reshape(NC, NS, NCHUNK, CHUNK)
    zeros = jnp.zeros((RPT, DH), jnp.float32)
    agg1 = _agg_from_out(_sc_segment_sum(_split_rows(x), src2, dst, zeros))
    h1 = _mlp1(x, agg1, W1, b1.reshape(1, D), W2, b2.reshape(1, D))
    agg2 = _agg_from_out(_sc_segment_sum(_split_rows(h1), src2, dst, zeros))
    return _mlp2(h1, agg2, W3, b3.reshape(1, D), W4, b4.reshape(1, D))


# speedup vs baseline: 68.1158x; 1.1404x over previous
import functools

import jax
import jax.numpy as jnp
from jax import lax
from jax.experimental import pallas as pl
from jax.experimental.pallas import tpu as pltpu
from jax.experimental.pallas import tpu_sc as plsc

N = 10000
E = 320000
D = 128
NC = 2
NS = 16
DH = D // NC
EPT = E // NS
CHUNK = 100
NCHUNK = EPT // CHUNK
NBUF = 4
RPT = N // NS

_sc_mesh = plsc.VectorSubcoreMesh(core_axis_name="c", subcore_axis_name="s")


@functools.partial(
    pl.kernel,
    out_type=jax.ShapeDtypeStruct((NC, NS, RPT, DH), jnp.float32),
    mesh=_sc_mesh,
    compiler_params=pltpu.CompilerParams(use_tc_tiling_on_sc=False),
    scratch_types=[
        pltpu.VMEM((NCHUNK, CHUNK), jnp.int32),
        pltpu.VMEM((NCHUNK, CHUNK), jnp.int32),
        pltpu.VMEM((NBUF, CHUNK, DH), jnp.float32),
        pltpu.VMEM_SHARED((N, DH), jnp.float32),
        [pltpu.SemaphoreType.DMA] * NBUF,
        [pltpu.SemaphoreType.DMA] * NBUF,
    ],
)
def _sc_segment_sum(x_hbm, src_hbm, dst_hbm, zeros_hbm, out_hbm,
                    src_v, dst_v, rows_v, acc, gsem, ssem):
    cid = lax.axis_index("c")
    sid = lax.axis_index("s")

    pltpu.sync_copy(zeros_hbm, acc.at[pl.ds(sid * RPT, RPT)])
    pltpu.sync_copy(src_hbm.at[cid, sid], src_v)
    pltpu.sync_copy(dst_hbm.at[sid], dst_v)
    plsc.subcore_barrier()

    for b in range(NBUF - 1):
        pltpu.async_copy(x_hbm.at[src_v.at[b]], rows_v.at[b], gsem[b])

    def body(jj, _):
        j0 = NBUF * jj
        for b in range(NBUF):
            j = j0 + b
            bp = (b + NBUF - 1) % NBUF

            @pl.when(j + NBUF - 1 < NCHUNK)
            def _prefetch():
                @pl.when(j >= 1)
                def _drain_prev_scatter():
                    pltpu.make_async_copy(rows_v.at[bp], acc.at[dst_v.at[0]],
                                          ssem[bp]).wait()
                pltpu.async_copy(x_hbm.at[src_v.at[j + NBUF - 1]],
                                 rows_v.at[bp], gsem[bp])

            pltpu.make_async_copy(x_hbm.at[src_v.at[j]], rows_v.at[b],
                                  gsem[b]).wait()
            pltpu.async_copy(rows_v.at[b], acc.at[dst_v.at[j]], ssem[b],
                             add=True)
        return 0

    lax.fori_loop(0, NCHUNK // NBUF, body, 0)
    for b in range(NBUF):
        pltpu.make_async_copy(rows_v.at[b], acc.at[dst_v.at[0]],
                              ssem[b]).wait()

    plsc.subcore_barrier()
    pltpu.sync_copy(acc.at[pl.ds(sid * RPT, RPT)], out_hbm.at[cid, sid])


def _mlp1_body(x_ref, agg_ref, w1_ref, b1_ref, w2_ref, b2_ref, o_ref):
    h = x_ref[...] + agg_ref[...]
    h = jnp.dot(h, w1_ref[...], preferred_element_type=jnp.float32) + b1_ref[...]
    h = jnp.maximum(h, 0.0)
    h = jnp.dot(h, w2_ref[...], preferred_element_type=jnp.float32) + b2_ref[...]
    o_ref[...] = jnp.maximum(h, 0.0)


def _mlp2_body(x_ref, agg_ref, w3_ref, b3_ref, w4_ref, b4_ref, o_ref):
    h = x_ref[...] + agg_ref[...]
    h = jnp.dot(h, w3_ref[...], preferred_element_type=jnp.float32) + b3_ref[...]
    h = jnp.maximum(h, 0.0)
    h = jnp.dot(h, w4_ref[...], preferred_element_type=jnp.float32) + b4_ref[...]
    m = jnp.max(h, axis=1, keepdims=True)
    s = jnp.sum(jnp.exp(h - m), axis=1, keepdims=True)
    o_ref[...] = h - m - jnp.log(s)


_mlp1 = pl.pallas_call(
    _mlp1_body, out_shape=jax.ShapeDtypeStruct((N, D), jnp.float32))
_mlp2 = pl.pallas_call(
    _mlp2_body, out_shape=jax.ShapeDtypeStruct((N, D), jnp.float32))


def _split_rows(h):
    return jnp.concatenate([h[:, :DH], h[:, DH:]], axis=0)


def _agg_from_out(out):
    o = out.reshape(NC, N, DH)
    return jnp.concatenate([o[0], o[1]], axis=1)


@jax.jit
def kernel(x, edge_index, W1, b1, W2, b2, W3, b3, W4, b4):
    src = edge_index[0]
    dst = edge_index[1].reshape(NS, NCHUNK, CHUNK)
    src2 = jnp.stack([src, src + N]).reshape(NC, NS, NCHUNK, CHUNK)
    zeros = jnp.zeros((RPT, DH), jnp.float32)
    agg1 = _agg_from_out(_sc_segment_sum(_split_rows(x), src2, dst, zeros))
    h1 = _mlp1(x, agg1, W1, b1.reshape(1, D), W2, b2.reshape(1, D))
    agg2 = _agg_from_out(_sc_segment_sum(_split_rows(h1), src2, dst, zeros))
    return _mlp2(h1, agg2, W3, b3.reshape(1, D), W4, b4.reshape(1, D))


# speedup vs baseline: 79.4581x; 1.1665x over previous
import functools

import jax
import jax.numpy as jnp
from jax import lax
from jax.experimental import pallas as pl
from jax.experimental.pallas import tpu as pltpu
from jax.experimental.pallas import tpu_sc as plsc

N = 10000
E = 320000
D = 128
NC = 2
NS = 16
DH = D // NC
EPT = E // NS
CHUNK = 100
NCHUNK = EPT // CHUNK
NBUF = 4
RPT = N // NS

_sc_mesh = plsc.VectorSubcoreMesh(core_axis_name="c", subcore_axis_name="s")


@functools.partial(
    pl.kernel,
    out_type=jax.ShapeDtypeStruct((NC, NS, RPT, DH), jnp.float32),
    mesh=_sc_mesh,
    compiler_params=pltpu.CompilerParams(use_tc_tiling_on_sc=False),
    scratch_types=[
        pltpu.VMEM((NCHUNK, CHUNK), jnp.int32),
        pltpu.VMEM((NCHUNK, CHUNK), jnp.int32),
        pltpu.VMEM((NBUF, CHUNK, DH), jnp.float32),
        pltpu.VMEM_SHARED((N, DH), jnp.float32),
        [pltpu.SemaphoreType.DMA] * NBUF,
        [pltpu.SemaphoreType.DMA] * NBUF,
    ],
)
def _sc_segment_sum(x_hbm, src_hbm, dst_hbm, zeros_hbm, out_hbm,
                    src_v, dst_v, rows_v, acc, gsem, ssem):
    cid = lax.axis_index("c")
    sid = lax.axis_index("s")

    pltpu.sync_copy(zeros_hbm, acc.at[pl.ds(sid * RPT, RPT)])
    pltpu.sync_copy(src_hbm.at[cid, sid], src_v)
    pltpu.sync_copy(dst_hbm.at[sid], dst_v)
    plsc.subcore_barrier()

    for b in range(NBUF - 1):
        pltpu.async_copy(x_hbm.at[src_v.at[b]], rows_v.at[b], gsem[b])

    def body(jj, _):
        j0 = NBUF * jj
        for b in range(NBUF):
            j = j0 + b
            bp = (b + NBUF - 1) % NBUF

            @pl.when(j + NBUF - 1 < NCHUNK)
            def _prefetch():
                @pl.when(j >= 1)
                def _drain_prev_scatter():
                    pltpu.make_async_copy(rows_v.at[bp], acc.at[dst_v.at[0]],
                                          ssem[bp]).wait()
                pltpu.async_copy(x_hbm.at[src_v.at[j + NBUF - 1]],
                                 rows_v.at[bp], gsem[bp])

            pltpu.make_async_copy(x_hbm.at[src_v.at[j]], rows_v.at[b],
                                  gsem[b]).wait()
            pltpu.async_copy(rows_v.at[b], acc.at[dst_v.at[j]], ssem[b],
                             add=True)
        return 0

    lax.fori_loop(0, NCHUNK // NBUF, body, 0)
    for b in range(NBUF):
        pltpu.make_async_copy(rows_v.at[b], acc.at[dst_v.at[0]],
                              ssem[b]).wait()

    plsc.subcore_barrier()
    pltpu.sync_copy(acc.at[pl.ds(sid * RPT, RPT)], out_hbm.at[cid, sid])


def _mlp1_body(x_ref, agg_ref, w1_ref, b1_ref, w2_ref, b2_ref,
               o_ref, osplit_ref):
    h = x_ref[...] + jnp.concatenate([agg_ref[0], agg_ref[1]], axis=1)
    h = jnp.dot(h, w1_ref[...], preferred_element_type=jnp.float32) + b1_ref[...]
    h = jnp.maximum(h, 0.0)
    h = jnp.dot(h, w2_ref[...], preferred_element_type=jnp.float32) + b2_ref[...]
    h = jnp.maximum(h, 0.0)
    o_ref[...] = h
    osplit_ref[0] = h[:, :DH]
    osplit_ref[1] = h[:, DH:]


def _mlp2_body(x_ref, agg_ref, w3_ref, b3_ref, w4_ref, b4_ref, o_ref):
    h = x_ref[...] + jnp.concatenate([agg_ref[0], agg_ref[1]], axis=1)
    h = jnp.dot(h, w3_ref[...], preferred_element_type=jnp.float32) + b3_ref[...]
    h = jnp.maximum(h, 0.0)
    h = jnp.dot(h, w4_ref[...], preferred_element_type=jnp.float32) + b4_ref[...]
    m = jnp.max(h, axis=1, keepdims=True)
    s = jnp.sum(jnp.exp(h - m), axis=1, keepdims=True)
    o_ref[...] = h - m - jnp.log(s)


_mlp1 = pl.pallas_call(
    _mlp1_body,
    out_shape=[jax.ShapeDtypeStruct((N, D), jnp.float32),
               jax.ShapeDtypeStruct((NC, N, DH), jnp.float32)])
_mlp2 = pl.pallas_call(
    _mlp2_body, out_shape=jax.ShapeDtypeStruct((N, D), jnp.float32))


def _split_rows(h):
    return jnp.concatenate([h[:, :DH], h[:, DH:]], axis=0)


@jax.jit
def kernel(x, edge_index, W1, b1, W2, b2, W3, b3, W4, b4):
    src = edge_index[0]
    dst = edge_index[1].reshape(NS, NCHUNK, CHUNK)
    src2 = jnp.stack([src, src + N]).reshape(NC, NS, NCHUNK, CHUNK)
    zeros = jnp.zeros((RPT, DH), jnp.float32)
    agg1 = _sc_segment_sum(_split_rows(x), src2, dst, zeros).reshape(NC, N, DH)
    h1, h1s = _mlp1(x, agg1, W1, b1.reshape(1, D), W2, b2.reshape(1, D))
    agg2 = _sc_segment_sum(h1s.reshape(NC * N, DH), src2, dst,
                           zeros).reshape(NC, N, DH)
    return _mlp2(h1, agg2, W3, b3.reshape(1, D), W4, b4.reshape(1, D))


# speedup vs baseline: 91.7230x; 1.1544x over previous
import functools

import jax
import jax.numpy as jnp
from jax import lax
from jax.experimental import pallas as pl
from jax.experimental.pallas import tpu as pltpu
from jax.experimental.pallas import tpu_sc as plsc

N = 10000
E = 320000
D = 128
NC = 2
NS = 16
DH = D // NC
EPT = E // NS
CHUNK = 100
NCHUNK = EPT // CHUNK
NBUF = 4
RPT = N // NS

_sc_mesh = plsc.VectorSubcoreMesh(core_axis_name="c", subcore_axis_name="s")


@functools.partial(
    pl.kernel,
    out_type=jax.ShapeDtypeStruct((NC, NS, RPT, DH), jnp.float32),
    mesh=_sc_mesh,
    compiler_params=pltpu.CompilerParams(use_tc_tiling_on_sc=False),
    scratch_types=[
        pltpu.VMEM((NCHUNK, CHUNK), jnp.int32),
        pltpu.VMEM((NCHUNK, CHUNK), jnp.int32),
        pltpu.VMEM((NBUF, CHUNK, DH), jnp.float32),
        pltpu.VMEM_SHARED((N, DH), jnp.float32),
        [pltpu.SemaphoreType.DMA] * NBUF,
        [pltpu.SemaphoreType.DMA] * NBUF,
    ],
)
def _sc_segment_sum(x_hbm, src_hbm, dst_hbm, zeros_hbm, out_hbm,
                    src_v, dst_v, rows_v, acc, gsem, ssem):
    cid = lax.axis_index("c")
    sid = lax.axis_index("s")

    pltpu.sync_copy(zeros_hbm, acc.at[pl.ds(sid * RPT, RPT)])
    pltpu.sync_copy(src_hbm.at[cid, sid], src_v)
    pltpu.sync_copy(dst_hbm.at[sid], dst_v)
    plsc.subcore_barrier()

    for b in range(NBUF - 1):
        pltpu.async_copy(x_hbm.at[src_v.at[b]], rows_v.at[b], gsem[b])

    def body(jj, _):
        j0 = NBUF * jj
        for b in range(NBUF):
            j = j0 + b
            bp = (b + NBUF - 1) % NBUF

            @pl.when(j + NBUF - 1 < NCHUNK)
            def _prefetch():
                @pl.when(j >= 1)
                def _drain_prev_scatter():
                    pltpu.make_async_copy(rows_v.at[bp], acc.at[dst_v.at[0]],
                                          ssem[bp]).wait()
                pltpu.async_copy(x_hbm.at[src_v.at[j + NBUF - 1]],
                                 rows_v.at[bp], gsem[bp])

            pltpu.make_async_copy(x_hbm.at[src_v.at[j]], rows_v.at[b],
                                  gsem[b]).wait()
            pltpu.async_copy(rows_v.at[b], acc.at[dst_v.at[j]], ssem[b],
                             add=True)
        return 0

    lax.fori_loop(0, NCHUNK // NBUF, body, 0)
    for b in range(NBUF):
        pltpu.make_async_copy(rows_v.at[b], acc.at[dst_v.at[0]],
                              ssem[b]).wait()

    plsc.subcore_barrier()
    pltpu.sync_copy(acc.at[pl.ds(sid * RPT, RPT)], out_hbm.at[cid, sid])


def _mlp1_body(x_ref, agg_ref, w1_ref, b1_ref, w2_ref, b2_ref, o_ref):
    h = x_ref[...] + jnp.concatenate([agg_ref[0], agg_ref[1]], axis=1)
    h = jnp.dot(h, w1_ref[...], preferred_element_type=jnp.float32) + b1_ref[...]
    h = jnp.maximum(h, 0.0)
    h = jnp.dot(h, w2_ref[...], preferred_element_type=jnp.float32) + b2_ref[...]
    o_ref[...] = jnp.maximum(h, 0.0)


def _mlp2_body(x_ref, agg_ref, w3_ref, b3_ref, w4_ref, b4_ref, o_ref):
    h = x_ref[...] + jnp.concatenate([agg_ref[0], agg_ref[1]], axis=1)
    h = jnp.dot(h, w3_ref[...], preferred_element_type=jnp.float32) + b3_ref[...]
    h = jnp.maximum(h, 0.0)
    h = jnp.dot(h, w4_ref[...], preferred_element_type=jnp.float32) + b4_ref[...]
    m = jnp.max(h, axis=1, keepdims=True)
    s = jnp.sum(jnp.exp(h - m), axis=1, keepdims=True)
    o_ref[...] = h - m - jnp.log(s)


_mlp1 = pl.pallas_call(
    _mlp1_body, out_shape=jax.ShapeDtypeStruct((N, D), jnp.float32))
_mlp2 = pl.pallas_call(
    _mlp2_body, out_shape=jax.ShapeDtypeStruct((N, D), jnp.float32))


@jax.jit
def kernel(x, edge_index, W1, b1, W2, b2, W3, b3, W4, b4):
    src = edge_index[0]
    dst = edge_index[1].reshape(NS, NCHUNK, CHUNK)
    src2 = jnp.stack([2 * src, 2 * src + 1]).reshape(NC, NS, NCHUNK, CHUNK)
    zeros = jnp.zeros((RPT, DH), jnp.float32)
    agg1 = _sc_segment_sum(x.reshape(NC * N, DH), src2, dst,
                           zeros).reshape(NC, N, DH)
    h1 = _mlp1(x, agg1, W1, b1.reshape(1, D), W2, b2.reshape(1, D))
    agg2 = _sc_segment_sum(h1.reshape(NC * N, DH), src2, dst,
                           zeros).reshape(NC, N, DH)
    return _mlp2(h1, agg2, W3, b3.reshape(1, D), W4, b4.reshape(1, D))


# speedup vs baseline: 92.2109x; 1.0053x over previous
import functools

import jax
import jax.numpy as jnp
from jax import lax
from jax.experimental import pallas as pl
from jax.experimental.pallas import tpu as pltpu
from jax.experimental.pallas import tpu_sc as plsc

N = 10000
E = 320000
D = 128
NC = 2
NS = 16
DH = D // NC
EPT = E // NS
CHUNK = 100
NCHUNK = EPT // CHUNK
NBUF = 5
RPT = N // NS

_sc_mesh = plsc.VectorSubcoreMesh(core_axis_name="c", subcore_axis_name="s")


@functools.partial(
    pl.kernel,
    out_type=jax.ShapeDtypeStruct((NC, NS, RPT, DH), jnp.float32),
    mesh=_sc_mesh,
    compiler_params=pltpu.CompilerParams(use_tc_tiling_on_sc=False),
    scratch_types=[
        pltpu.VMEM((NCHUNK, CHUNK), jnp.int32),
        pltpu.VMEM((NCHUNK, CHUNK), jnp.int32),
        pltpu.VMEM((NBUF, CHUNK, DH), jnp.float32),
        pltpu.VMEM_SHARED((N, DH), jnp.float32),
        [pltpu.SemaphoreType.DMA] * NBUF,
        [pltpu.SemaphoreType.DMA] * NBUF,
    ],
)
def _sc_segment_sum(x_hbm, src_hbm, dst_hbm, zeros_hbm, out_hbm,
                    src_v, dst_v, rows_v, acc, gsem, ssem):
    cid = lax.axis_index("c")
    sid = lax.axis_index("s")

    pltpu.sync_copy(zeros_hbm, acc.at[pl.ds(sid * RPT, RPT)])
    pltpu.sync_copy(src_hbm.at[cid, sid], src_v)
    pltpu.sync_copy(dst_hbm.at[sid], dst_v)
    plsc.subcore_barrier()

    for b in range(NBUF - 1):
        pltpu.async_copy(x_hbm.at[src_v.at[b]], rows_v.at[b], gsem[b])

    def body(jj, _):
        j0 = NBUF * jj
        for b in range(NBUF):
            j = j0 + b
            bp = (b + NBUF - 1) % NBUF

            @pl.when(j + NBUF - 1 < NCHUNK)
            def _prefetch():
                @pl.when(j >= 1)
                def _drain_prev_scatter():
                    pltpu.make_async_copy(rows_v.at[bp], acc.at[dst_v.at[0]],
                                          ssem[bp]).wait()
                pltpu.async_copy(x_hbm.at[src_v.at[j + NBUF - 1]],
                                 rows_v.at[bp], gsem[bp])

            pltpu.make_async_copy(x_hbm.at[src_v.at[j]], rows_v.at[b],
                                  gsem[b]).wait()
            pltpu.async_copy(rows_v.at[b], acc.at[dst_v.at[j]], ssem[b],
                             add=True)
        return 0

    lax.fori_loop(0, NCHUNK // NBUF, body, 0)
    for b in range(NBUF):
        pltpu.make_async_copy(rows_v.at[b], acc.at[dst_v.at[0]],
                              ssem[b]).wait()

    plsc.subcore_barrier()
    pltpu.sync_copy(acc.at[pl.ds(sid * RPT, RPT)], out_hbm.at[cid, sid])


def _mlp1_body(x_ref, agg_ref, w1_ref, b1_ref, w2_ref, b2_ref, o_ref):
    h = x_ref[...] + jnp.concatenate([agg_ref[0], agg_ref[1]], axis=1)
    h = jnp.dot(h, w1_ref[...], preferred_element_type=jnp.float32) + b1_ref[...]
    h = jnp.maximum(h, 0.0)
    h = jnp.dot(h, w2_ref[...], preferred_element_type=jnp.float32) + b2_ref[...]
    o_ref[...] = jnp.maximum(h, 0.0)


def _mlp2_body(x_ref, agg_ref, w3_ref, b3_ref, w4_ref, b4_ref, o_ref):
    h = x_ref[...] + jnp.concatenate([agg_ref[0], agg_ref[1]], axis=1)
    h = jnp.dot(h, w3_ref[...], preferred_element_type=jnp.float32) + b3_ref[...]
    h = jnp.maximum(h, 0.0)
    h = jnp.dot(h, w4_ref[...], preferred_element_type=jnp.float32) + b4_ref[...]
    m = jnp.max(h, axis=1, keepdims=True)
    s = jnp.sum(jnp.exp(h - m), axis=1, keepdims=True)
    o_ref[...] = h - m - jnp.log(s)


_mlp1 = pl.pallas_call(
    _mlp1_body, out_shape=jax.ShapeDtypeStruct((N, D), jnp.float32))
_mlp2 = pl.pallas_call(
    _mlp2_body, out_shape=jax.ShapeDtypeStruct((N, D), jnp.float32))


@jax.jit
def kernel(x, edge_index, W1, b1, W2, b2, W3, b3, W4, b4):
    src = edge_index[0]
    dst = edge_index[1].reshape(NS, NCHUNK, CHUNK)
    src2 = jnp.stack([2 * src, 2 * src + 1]).reshape(NC, NS, NCHUNK, CHUNK)
    zeros = jnp.zeros((RPT, DH), jnp.float32)
    agg1 = _sc_segment_sum(x.reshape(NC * N, DH), src2, dst,
                           zeros).reshape(NC, N, DH)
    h1 = _mlp1(x, agg1, W1, b1.reshape(1, D), W2, b2.reshape(1, D))
    agg2 = _sc_segment_sum(h1.reshape(NC * N, DH), src2, dst,
                           zeros).reshape(NC, N, DH)
    return _mlp2(h1, agg2, W3, b3.reshape(1, D), W4, b4.reshape(1, D))
